# baseline (speedup 1.0000x reference)
import jax
import jax.numpy as jnp
from jax import lax
from jax.experimental import pallas as pl
from jax.experimental.pallas import tpu as pltpu
from jax.experimental.pallas import tpu_sc as plsc

HIDDEN = 128
MAX_POS = 1000
NUM_K = 4

_info = plsc.get_sparse_core_info()
_NC, _NS, _L = _info.num_cores, _info.num_subcores, _info.num_lanes
_NW = _NC * _NS

N_ROWS = 1024 * 200
ROWS_PER_W = N_ROWS // _NW
CHUNK = 128
N_CHUNKS = ROWS_PER_W // CHUNK
N_PAIRS = N_CHUNKS // 2


SLOTS = 4
N_FULL_ROUNDS = N_CHUNKS // SLOTS
N_LEFTOVER = N_CHUNKS - N_FULL_ROUNDS * SLOTS


def _sc_body(xt_hbm, yt_hbm, ht_hbm, wt_hbm, bboxw_hbm, out_hbm,
             idx_all, sh_tab, outv0, outv1, outv2, outv3,
             gsem0, gsem1, gsem2, gsem3,
             osem0, osem1, osem2, osem3):
    sid = lax.axis_index("s")
    wid = sid * _NC + lax.axis_index("c")
    base_row_w = wid * ROWS_PER_W
    hbm_tabs = (xt_hbm, yt_hbm, ht_hbm, wt_hbm)
    outs = (outv0, outv1, outv2, outv3)
    gsems = (gsem0, gsem1, gsem2, gsem3)
    osems = (osem0, osem1, osem2, osem3)

    @pl.when(sid == 0)
    def _():
        for k in range(NUM_K):
            pltpu.sync_copy(hbm_tabs[k],
                            sh_tab.at[pl.ds(k * MAX_POS, MAX_POS)])

    pltpu.sync_copy(bboxw_hbm.at[wid], idx_all)
    plsc.subcore_barrier()
    sh_tabs = tuple(sh_tab.at[pl.ds(k * MAX_POS, MAX_POS)]
                    for k in range(NUM_K))
    tabs_by_slot = (sh_tabs, sh_tabs, sh_tabs, hbm_tabs)

    def fire(c, b, k):
        pltpu.async_copy(tabs_by_slot[b][k].at[idx_all.at[k * N_CHUNKS + c]],
                         outs[b], gsems[b], add=(k > 0))

    def gather_wait(b):
        pltpu.make_async_copy(tabs_by_slot[b][0].at[idx_all.at[0]], outs[b],
                              gsems[b]).wait()

    def store(c, b):
        row_base = base_row_w + c * CHUNK
        pltpu.async_copy(outs[b], out_hbm.at[pl.ds(row_base, CHUNK)],
                         osems[b])

    def store_wait(b):
        pltpu.make_async_copy(outs[b], out_hbm.at[pl.ds(0, CHUNK)],
                              osems[b]).wait()

    def round_(p, with_store_wait, nslots=SLOTS):
        cs = [p * SLOTS + b for b in range(nslots)]
        if with_store_wait:
            for b in range(nslots):
                store_wait(b)
        for b in range(nslots):
            fire(cs[b], b, 0)
        for k in range(1, NUM_K):
            for b in range(nslots):
                gather_wait(b)
                fire(cs[b], b, k)
        for b in range(nslots):
            gather_wait(b)
            store(cs[b], b)

    round_(0, False)

    def round_body(p, _):
        round_(p, True)
        return 0

    lax.fori_loop(1, N_FULL_ROUNDS, round_body, 0, unroll=False)

    if N_LEFTOVER:
        round_(N_FULL_ROUNDS, True, nslots=N_LEFTOVER)

    for b in range(SLOTS):
        store_wait(b)


@jax.jit
def kernel(bbox, x_table, y_table, h_table, w_table):
    bboxw = (bbox.astype(jnp.int32)
             .transpose(2, 0, 1)
             .reshape(NUM_K, _NW, N_CHUNKS, CHUNK)
             .transpose(1, 0, 2, 3)
             .reshape(_NW, NUM_K * N_CHUNKS, CHUNK))

    mesh = plsc.VectorSubcoreMesh(core_axis_name="c", subcore_axis_name="s")
    run = pl.kernel(
        _sc_body,
        out_type=jax.ShapeDtypeStruct((N_ROWS, HIDDEN), jnp.float32),
        mesh=mesh,
        scratch_types=[
            pltpu.VMEM((NUM_K * N_CHUNKS, CHUNK), jnp.int32),
            pltpu.VMEM_SHARED((NUM_K * MAX_POS, HIDDEN), jnp.float32),
        ] + [pltpu.VMEM((CHUNK, HIDDEN), jnp.float32)] * SLOTS
          + [pltpu.SemaphoreType.DMA] * (2 * SLOTS),
    )
    out = run(x_table, y_table, h_table, w_table, bboxw)
    return out.reshape(1024, 200, HIDDEN)


# speedup vs baseline: 1.0508x; 1.0508x over previous
import jax
import jax.numpy as jnp
from jax import lax
from jax.experimental import pallas as pl
from jax.experimental.pallas import tpu as pltpu
from jax.experimental.pallas import tpu_sc as plsc

HIDDEN = 128
MAX_POS = 1000
NUM_K = 4

_info = plsc.get_sparse_core_info()
_NC, _NS, _L = _info.num_cores, _info.num_subcores, _info.num_lanes
_NW = _NC * _NS

N_ROWS = 1024 * 200
ROWS_PER_W = N_ROWS // _NW
CHUNK = 128
N_CHUNKS = ROWS_PER_W // CHUNK
N_PAIRS = N_CHUNKS // 2


SLOTS = 4
N_FULL_ROUNDS = N_CHUNKS // SLOTS
N_LEFTOVER = N_CHUNKS - N_FULL_ROUNDS * SLOTS


def _sc_body(xt_hbm, yt_hbm, ht_hbm, wt_hbm, bboxw_hbm, out_hbm,
             idx_all, sh_tab, outv0, outv1, outv2, outv3,
             gsem0, gsem1, gsem2, gsem3,
             osem0, osem1, osem2, osem3):
    sid = lax.axis_index("s")
    wid = sid * _NC + lax.axis_index("c")
    base_row_w = wid * ROWS_PER_W
    hbm_tabs = (xt_hbm, yt_hbm, ht_hbm, wt_hbm)
    outs = (outv0, outv1, outv2, outv3)
    gsems = (gsem0, gsem1, gsem2, gsem3)
    osems = (osem0, osem1, osem2, osem3)

    @pl.when(sid == 0)
    def _():
        for k in range(NUM_K):
            pltpu.sync_copy(hbm_tabs[k],
                            sh_tab.at[pl.ds(k * MAX_POS, MAX_POS)])

    pltpu.sync_copy(bboxw_hbm.at[wid], idx_all)
    plsc.subcore_barrier()
    sh_tabs = tuple(sh_tab.at[pl.ds(k * MAX_POS, MAX_POS)]
                    for k in range(NUM_K))
    tabs_by_slot = (sh_tabs, sh_tabs, hbm_tabs, hbm_tabs)

    def fire(c, b, k):
        pltpu.async_copy(tabs_by_slot[b][k].at[idx_all.at[k * N_CHUNKS + c]],
                         outs[b], gsems[b], add=(k > 0))

    def gather_wait(b):
        pltpu.make_async_copy(tabs_by_slot[b][0].at[idx_all.at[0]], outs[b],
                              gsems[b]).wait()

    def store(c, b):
        row_base = base_row_w + c * CHUNK
        pltpu.async_copy(outs[b], out_hbm.at[pl.ds(row_base, CHUNK)],
                         osems[b])

    def store_wait(b):
        pltpu.make_async_copy(outs[b], out_hbm.at[pl.ds(0, CHUNK)],
                              osems[b]).wait()

    def round_(p, with_store_wait, nslots=SLOTS):
        cs = [p * SLOTS + b for b in range(nslots)]
        if with_store_wait:
            for b in range(nslots):
                store_wait(b)
        for b in range(nslots):
            fire(cs[b], b, 0)
        for k in range(1, NUM_K):
            for b in range(nslots):
                gather_wait(b)
                fire(cs[b], b, k)
        for b in range(nslots):
            gather_wait(b)
            store(cs[b], b)

    round_(0, False)

    def round_body(p, _):
        round_(p, True)
        return 0

    lax.fori_loop(1, N_FULL_ROUNDS, round_body, 0, unroll=False)

    if N_LEFTOVER:
        round_(N_FULL_ROUNDS, True, nslots=N_LEFTOVER)

    for b in range(SLOTS):
        store_wait(b)


@jax.jit
def kernel(bbox, x_table, y_table, h_table, w_table):
    bboxw = (bbox.astype(jnp.int32)
             .transpose(2, 0, 1)
             .reshape(NUM_K, _NW, N_CHUNKS, CHUNK)
             .transpose(1, 0, 2, 3)
             .reshape(_NW, NUM_K * N_CHUNKS, CHUNK))

    mesh = plsc.VectorSubcoreMesh(core_axis_name="c", subcore_axis_name="s")
    run = pl.kernel(
        _sc_body,
        out_type=jax.ShapeDtypeStruct((N_ROWS, HIDDEN), jnp.float32),
        mesh=mesh,
        scratch_types=[
            pltpu.VMEM((NUM_K * N_CHUNKS, CHUNK), jnp.int32),
            pltpu.VMEM_SHARED((NUM_K * MAX_POS, HIDDEN), jnp.float32),
        ] + [pltpu.VMEM((CHUNK, HIDDEN), jnp.float32)] * SLOTS
          + [pltpu.SemaphoreType.DMA] * (2 * SLOTS),
    )
    out = run(x_table, y_table, h_table, w_table, bboxw)
    return out.reshape(1024, 200, HIDDEN)
